# baseline (speedup 1.0000x reference)
import functools

import jax
import jax.numpy as jnp
from jax import lax
from jax.experimental import pallas as pl
from jax.experimental.pallas import tpu as pltpu
from jax.experimental.pallas import tpu_sc as plsc


def _transform_body(x_ref, w_ref, b_ref, g_ref, bt_ref, o_ref):
    h = jnp.dot(x_ref[...], w_ref[...], preferred_element_type=jnp.float32)
    h = h + b_ref[...]
    m = jnp.mean(h, axis=-1, keepdims=True)
    v = jnp.mean(jnp.square(h - m), axis=-1, keepdims=True)
    h = (h - m) * lax.rsqrt(v + 1e-5) * g_ref[...] + bt_ref[...]
    o_ref[...] = h * jax.nn.sigmoid(h)


def _transform(xf, W, b, gamma, beta, blk):
    R, H = xf.shape
    assert R % blk == 0
    return pl.pallas_call(
        _transform_body,
        grid=(R // blk,),
        in_specs=[
            pl.BlockSpec((blk, H), lambda i: (i, 0)),
            pl.BlockSpec((H, H), lambda i: (0, 0)),
            pl.BlockSpec((1, H), lambda i: (0, 0)),
            pl.BlockSpec((1, H), lambda i: (0, 0)),
            pl.BlockSpec((1, H), lambda i: (0, 0)),
        ],
        out_specs=pl.BlockSpec((blk, H), lambda i: (i, 0)),
        out_shape=jax.ShapeDtypeStruct((R, H), jnp.float32),
    )(xf, W, b.reshape(1, H), gamma.reshape(1, H), beta.reshape(1, H))


_CHUNK = 128
_NW = 32


_FCHUNK = _CHUNK // 2


@functools.lru_cache(maxsize=None)
def _make_gather(T, rows, cpw):
    full_chunks = rows // _CHUNK
    rem = rows % _CHUNK
    n_work = full_chunks + (1 if rem else 0)

    mesh = plsc.VectorSubcoreMesh(core_axis_name="c", subcore_axis_name="s")

    assert cpw >= 4 and cpw % 3 == 1
    c_fast, c_slow = cpw + 3, cpw - 3

    @functools.partial(
        pl.kernel,
        mesh=mesh,
        out_type=jax.ShapeDtypeStruct((rows, 128), jnp.float32),
        scratch_types=[
            pltpu.VMEM(((cpw + 3) * _FCHUNK + 16,), jnp.int32),
            pltpu.VMEM((3, _CHUNK), jnp.int32),
            pltpu.VMEM((_CHUNK, 128), jnp.float32),
            pltpu.VMEM((_CHUNK, 128), jnp.float32),
            pltpu.VMEM((_CHUNK, 128), jnp.float32),
            pltpu.SemaphoreType.DMA,
            pltpu.SemaphoreType.DMA,
            pltpu.SemaphoreType.DMA,
            pltpu.SemaphoreType.DMA,
            pltpu.SemaphoreType.DMA,
            pltpu.SemaphoreType.DMA,
        ],
    )
    def gather(table_hbm, idx_hbm, out_hbm, idx_all, idx_v,
               rows_0, rows_1, rows_2, g0, g1, g2, s0, s1, s2):
        cc = lax.axis_index("c")
        ss = lax.axis_index("s")
        rows_t = (rows_0, rows_1, rows_2)
        gsem = (g0, g1, g2)
        ssem = (s0, s1, s2)

        iot = lax.iota(jnp.int32, 16)
        dup = iot >> 1
        par = iot & 1

        def run(base_chunk, cpwx):
            pltpu.sync_copy(
                idx_hbm.at[pl.ds(base_chunk * _FCHUNK, cpwx * _FCHUNK)],
                idx_all.at[pl.ds(0, cpwx * _FCHUNK)])

            def fire_gather(k, t):
                base = pl.multiple_of(k * _FCHUNK, _FCHUNK)
                for j in range(_CHUNK // 16):
                    v = idx_all[pl.ds(base + j * 8, 16)]
                    g = v.at[dup].get(mode="promise_in_bounds")
                    idx_v[t, pl.ds(j * 16, 16)] = g * 2 + par
                pltpu.async_copy(table_hbm.at[idx_v.at[t]], rows_t[t], gsem[t])

            def _store_copies(k, t):
                chunk = base_chunk + k
                full = pltpu.make_async_copy(
                    rows_t[t], out_hbm.at[pl.ds(chunk * _CHUNK, _CHUNK)], ssem[t])
                part = pltpu.make_async_copy(
                    rows_t[t].at[pl.ds(0, rem)],
                    out_hbm.at[pl.ds(full_chunks * _CHUNK, rem)], ssem[t]) if rem else None
                return chunk, full, part

            def drain_gather_fire_store(k, t):
                pltpu.make_async_copy(table_hbm.at[idx_v.at[t]], rows_t[t], gsem[t]).wait()
                chunk, full, part = _store_copies(k, t)

                @pl.when(chunk < full_chunks)
                def _():
                    full.start()

                if rem:
                    @pl.when(chunk == full_chunks)
                    def _():
                        part.start()

            def drain_store(k, t):
                chunk, full, part = _store_copies(k, t)

                @pl.when(chunk < full_chunks)
                def _():
                    full.wait()

                if rem:
                    @pl.when(chunk == full_chunks)
                    def _():
                        part.wait()

            fire_gather(0, 0)
            fire_gather(1, 1)
            drain_gather_fire_store(0, 0)
            fire_gather(2, 2)

            def body(j, carry):
                for dk in range(3):
                    k = 3 * j + 1 + dk
                    t = (1 + dk) % 3
                    drain_gather_fire_store(k, t)
                    drain_store(k - 1, (t + 2) % 3)
                    fire_gather(k + 2, (t + 2) % 3)
                return carry

            lax.fori_loop(0, (cpwx - 4) // 3, body, 0)
            for k in (cpwx - 3, cpwx - 2, cpwx - 1):
                drain_gather_fire_store(k, k % 3)
                drain_store(k - 1, (k - 1) % 3)
                if k + 2 <= cpwx - 1:
                    fire_gather(k + 2, (k + 2) % 3)
            drain_store(cpwx - 1, (cpwx - 1) % 3)

        @pl.when(cc == 0)
        def _():
            run(ss * c_fast, c_fast)

        @pl.when(cc == 1)
        def _():
            run(16 * c_fast + ss * c_slow, c_slow)

    return gather


def kernel(x, hierarchy_mapping, num_fine_nodes, W, b, gamma, beta):
    B, NC, Fm, H = x.shape
    NF = hierarchy_mapping.shape[0]
    assert H == 128 and Fm == 2

    xtf = _transform(x.reshape(-1, H), W, b, gamma, beta, blk=10000)

    fidx = (hierarchy_mapping[None, :].astype(jnp.int32)
            + NC * jnp.arange(B, dtype=jnp.int32)[:, None]).reshape(-1)
    rows = B * NF * Fm
    n_chunks = -(-rows // _CHUNK)
    cpw = -(-n_chunks // _NW)
    pad = _NW * cpw * _FCHUNK - fidx.shape[0]
    if pad:
        fidx = jnp.concatenate([fidx, jnp.zeros((pad,), jnp.int32)])

    out = _make_gather(B * NC * Fm, rows, cpw)(xtf, fidx)
    return out.reshape(B, NF, Fm, H)


# speedup vs baseline: 1.0159x; 1.0159x over previous
import functools

import jax
import jax.numpy as jnp
from jax import lax
from jax.experimental import pallas as pl
from jax.experimental.pallas import tpu as pltpu
from jax.experimental.pallas import tpu_sc as plsc


def _transform_body(x_ref, w_ref, b_ref, g_ref, bt_ref, o_ref):
    h = jnp.dot(x_ref[...], w_ref[...], preferred_element_type=jnp.float32)
    h = h + b_ref[...]
    m = jnp.mean(h, axis=-1, keepdims=True)
    v = jnp.mean(jnp.square(h - m), axis=-1, keepdims=True)
    h = (h - m) * lax.rsqrt(v + 1e-5) * g_ref[...] + bt_ref[...]
    o_ref[...] = h * jax.nn.sigmoid(h)


def _transform(xf, W, b, gamma, beta, blk):
    R, H = xf.shape
    assert R % blk == 0
    return pl.pallas_call(
        _transform_body,
        grid=(R // blk,),
        in_specs=[
            pl.BlockSpec((blk, H), lambda i: (i, 0)),
            pl.BlockSpec((H, H), lambda i: (0, 0)),
            pl.BlockSpec((1, H), lambda i: (0, 0)),
            pl.BlockSpec((1, H), lambda i: (0, 0)),
            pl.BlockSpec((1, H), lambda i: (0, 0)),
        ],
        out_specs=pl.BlockSpec((blk, H), lambda i: (i, 0)),
        out_shape=jax.ShapeDtypeStruct((R, H), jnp.float32),
    )(xf, W, b.reshape(1, H), gamma.reshape(1, H), beta.reshape(1, H))


_CHUNK = 128
_NW = 32


_FCHUNK = _CHUNK // 2


@functools.lru_cache(maxsize=None)
def _make_gather(T, rows, cpw):
    full_chunks = rows // _CHUNK
    rem = rows % _CHUNK
    n_work = full_chunks + (1 if rem else 0)

    mesh = plsc.VectorSubcoreMesh(core_axis_name="c", subcore_axis_name="s")

    assert cpw >= 4 and cpw % 3 == 1

    @functools.partial(
        pl.kernel,
        mesh=mesh,
        out_type=jax.ShapeDtypeStruct((rows, 128), jnp.float32),
        scratch_types=[
            pltpu.VMEM((cpw * _FCHUNK + 16,), jnp.int32),
            pltpu.VMEM((3, _CHUNK), jnp.int32),
            pltpu.VMEM((_CHUNK, 128), jnp.float32),
            pltpu.VMEM((_CHUNK, 128), jnp.float32),
            pltpu.VMEM((_CHUNK, 128), jnp.float32),
            pltpu.SemaphoreType.DMA,
            pltpu.SemaphoreType.DMA,
            pltpu.SemaphoreType.DMA,
            pltpu.SemaphoreType.DMA,
            pltpu.SemaphoreType.DMA,
            pltpu.SemaphoreType.DMA,
        ],
    )
    def gather(table_hbm, idx_hbm, out_hbm, idx_all, idx_v,
               rows_0, rows_1, rows_2, g0, g1, g2, s0, s1, s2):
        wid = lax.axis_index("s") * 2 + lax.axis_index("c")
        rows_t = (rows_0, rows_1, rows_2)
        gsem = (g0, g1, g2)
        ssem = (s0, s1, s2)

        pltpu.sync_copy(idx_hbm.at[pl.ds(wid * (cpw * _FCHUNK), cpw * _FCHUNK)],
                        idx_all.at[pl.ds(0, cpw * _FCHUNK)])

        iot = lax.iota(jnp.int32, 16)
        dup = iot >> 1
        par = iot & 1

        def fire_gather(k, t):
            base = pl.multiple_of(k * _FCHUNK, _FCHUNK)
            for j in range(_CHUNK // 16):
                v = idx_all[pl.ds(base + j * 8, 16)]
                g = v.at[dup].get(mode="promise_in_bounds")
                idx_v[t, pl.ds(j * 16, 16)] = g * 2 + par
            pltpu.async_copy(table_hbm.at[idx_v.at[t]], rows_t[t], gsem[t])

        def _store_copies(k, t):
            chunk = wid * cpw + k
            full = pltpu.make_async_copy(
                rows_t[t], out_hbm.at[pl.ds(chunk * _CHUNK, _CHUNK)], ssem[t])
            part = pltpu.make_async_copy(
                rows_t[t].at[pl.ds(0, rem)],
                out_hbm.at[pl.ds(full_chunks * _CHUNK, rem)], ssem[t]) if rem else None
            return chunk, full, part

        def drain_gather_fire_store(k, t):
            pltpu.make_async_copy(table_hbm.at[idx_v.at[t]], rows_t[t], gsem[t]).wait()
            chunk, full, part = _store_copies(k, t)

            @pl.when(chunk < full_chunks)
            def _():
                full.start()

            if rem:
                @pl.when(chunk == full_chunks)
                def _():
                    part.start()

        def drain_store(k, t):
            chunk, full, part = _store_copies(k, t)

            @pl.when(chunk < full_chunks)
            def _():
                full.wait()

            if rem:
                @pl.when(chunk == full_chunks)
                def _():
                    part.wait()

        fire_gather(0, 0)
        fire_gather(1, 1)
        drain_gather_fire_store(0, 0)
        fire_gather(2, 2)

        def body(j, carry):
            for dk in range(3):
                k = 3 * j + 1 + dk
                t = (1 + dk) % 3
                drain_gather_fire_store(k, t)
                drain_store(k - 1, (t + 2) % 3)
                fire_gather(k + 2, (t + 2) % 3)
            return carry

        lax.fori_loop(0, (cpw - 4) // 3, body, 0)
        for k in (cpw - 3, cpw - 2, cpw - 1):
            drain_gather_fire_store(k, k % 3)
            drain_store(k - 1, (k - 1) % 3)
            if k + 2 <= cpw - 1:
                fire_gather(k + 2, (k + 2) % 3)
        drain_store(cpw - 1, (cpw - 1) % 3)

    return gather


def kernel(x, hierarchy_mapping, num_fine_nodes, W, b, gamma, beta):
    B, NC, Fm, H = x.shape
    NF = hierarchy_mapping.shape[0]
    assert H == 128 and Fm == 2

    xtf = _transform(x.reshape(-1, H), W, b, gamma, beta, blk=10000)

    fidx = (hierarchy_mapping[None, :].astype(jnp.int32)
            + NC * jnp.arange(B, dtype=jnp.int32)[:, None]).reshape(-1)
    rows = B * NF * Fm
    n_chunks = -(-rows // _CHUNK)
    cpw = -(-n_chunks // _NW)
    pad = _NW * cpw * _FCHUNK - fidx.shape[0]
    if pad:
        fidx = jnp.concatenate([fidx, jnp.zeros((pad,), jnp.int32)])

    out = _make_gather(B * NC * Fm, rows, cpw)(xtf, fidx)
    return out.reshape(B, NF, Fm, H)
